# baseline (speedup 1.0000x reference)
import functools

import jax
import jax.numpy as jnp
from jax import lax
from jax.experimental import pallas as pl
from jax.experimental.pallas import tpu as pltpu
from jax.experimental.pallas import tpu_sc as plsc

NC, NS = 2, 16
NW = NC * NS
CHUNK = 128
EPS = 1e-5


def _sc_gather(i0, i1, i2, i3, t0, t1, t2, t3, n_tokens, d):
    chunks = n_tokens // (NW * CHUNK)
    per_w = chunks * CHUNK
    mesh = plsc.VectorSubcoreMesh(
        core_axis_name="c", subcore_axis_name="s", num_cores=NC,
        num_subcores=NS)
    out_t = [jax.ShapeDtypeStruct((n_tokens, d), jnp.float32)] * 4
    scratch = (
        [pltpu.VMEM((chunks, CHUNK), jnp.int32) for _ in range(4)]
        + [pltpu.VMEM((CHUNK, d), jnp.float32) for _ in range(4)]
        + [pltpu.SemaphoreType.DMA]
    )

    @functools.partial(pl.kernel, out_type=out_t, mesh=mesh,
                       scratch_types=scratch,
                       compiler_params=pltpu.CompilerParams(
                           use_tc_tiling_on_sc=False))
    def k(ih0, ih1, ih2, ih3, th0, th1, th2, th3,
          o0, o1, o2, o3,
          iv0, iv1, iv2, iv3, rv0, rv1, rv2, rv3, sem):
        wid = lax.axis_index("s") * NC + lax.axis_index("c")
        base = wid * per_w
        ivs = (iv0, iv1, iv2, iv3)
        rvs = (rv0, rv1, rv2, rv3)
        ths = (th0, th1, th2, th3)
        outs = (o0, o1, o2, o3)
        for ih, iv in zip((ih0, ih1, ih2, ih3), ivs):
            pltpu.sync_copy(ih.at[wid], iv)

        def chunk_body(j, carry):
            cps = [pltpu.async_copy(th.at[iv.at[j]], rv, sem)
                   for th, iv, rv in zip(ths, ivs, rvs)]
            for cp in cps:
                cp.wait()
            for rv, ov in zip(rvs, outs):
                pltpu.sync_copy(rv, ov.at[pl.ds(base + j * CHUNK, CHUNK)])
            return carry

        lax.fori_loop(0, chunks, chunk_body, 0)

    return k(i0, i1, i2, i3, t0, t1, t2, t3)


def _tc_dense(e0, e1, e2, e3, el, wT, bc, wc, bct, g1, be1, g2, be2,
              n_tokens, hd, tile):
    grid = (n_tokens // tile,)
    intd = e0.shape[1]

    def body(e0r, e1r, e2r, e3r, elr, wTr, bcr, wcr, bctr, g1r, be1r,
             g2r, be2r, out):
        x = jnp.concatenate(
            [e0r[...], e1r[...], e2r[...], e3r[...]], axis=-1)
        X = lax.dot_general(x, wTr[...], (((1,), (0,)), ((), ())),
                            preferred_element_type=jnp.float32)
        X = X + bcr[...]
        mu = jnp.mean(X, axis=-1, keepdims=True)
        var = jnp.mean((X - mu) ** 2, axis=-1, keepdims=True)
        Xn = (X - mu) * lax.rsqrt(var + EPS) * g1r[...] + be1r[...]
        Y = elr[...] * wcr[...] + bctr[...]
        muY = jnp.mean(Y, axis=-1, keepdims=True)
        varY = jnp.mean((Y - muY) ** 2, axis=-1, keepdims=True)
        Yn = (Y - muY) * lax.rsqrt(varY + EPS) * g2r[...] + be2r[...]
        out[:, :hd] = Xn
        out[:, hd:] = Yn

    tok_spec = lambda w: pl.BlockSpec((tile, w), lambda i: (i, 0))
    fix_spec = lambda s: pl.BlockSpec(s, lambda i: (0, 0))
    return pl.pallas_call(
        body,
        grid=grid,
        in_specs=[tok_spec(intd)] * 4 + [tok_spec(1),
                  fix_spec((4 * intd, hd))] + [fix_spec((1, hd))] * 7,
        out_specs=pl.BlockSpec((tile, 2 * hd), lambda i: (i, 0)),
        out_shape=jax.ShapeDtypeStruct((n_tokens, 2 * hd), jnp.float32),
        compiler_params=pltpu.CompilerParams(
            dimension_semantics=("arbitrary",)),
    )(e0, e1, e2, e3, el, wT, bc, wc, bct, g1, be1, g2, be2)


def kernel(assessmentItemID, testId, KnowledgeTag, Interaction, elapsed,
           emb_item, emb_test, emb_tag, emb_inter,
           W_comb, b_comb, W_cont, b_cont,
           g_cat, beta_cat, g_cont, beta_cont):
    B, L = assessmentItemID.shape
    n = B * L
    intd = emb_item.shape[1]
    hd = W_comb.shape[0]
    chunks = n // (NW * CHUNK)
    slab = lambda a: a.reshape(NW, chunks, CHUNK)
    e0, e1, e2, e3 = _sc_gather(
        slab(assessmentItemID), slab(testId), slab(KnowledgeTag),
        slab(Interaction), emb_item, emb_test, emb_tag, emb_inter, n, intd)
    row = lambda v: v.reshape(1, hd)
    out = _tc_dense(
        e0, e1, e2, e3, elapsed.reshape(n, 1), W_comb.T, row(b_comb),
        W_cont.reshape(1, hd), row(b_cont), row(g_cat), row(beta_cat),
        row(g_cont), row(beta_cont), n, hd, tile=2048)
    return out.reshape(B, L, 2 * hd), B


# speedup vs baseline: 4.6142x; 4.6142x over previous
import functools

import jax
import jax.numpy as jnp
from jax import lax
from jax.experimental import pallas as pl
from jax.experimental.pallas import tpu as pltpu
from jax.experimental.pallas import tpu_sc as plsc

NC, NS = 2, 16
NW = NC * NS
IDXW = 128
SUB = 2
CHUNK = SUB * IDXW
EPS = 1e-5


def _sc_gather(i0, i1, i2, t0, t1, t2, n_tokens, d):
    per_w = n_tokens // NW
    idx_rows = per_w // IDXW
    chunks = per_w // CHUNK
    assert chunks % 2 == 1 and chunks >= 3
    mesh = plsc.VectorSubcoreMesh(
        core_axis_name="c", subcore_axis_name="s", num_cores=NC,
        num_subcores=NS)
    out_t = [jax.ShapeDtypeStruct((n_tokens, d), jnp.float32)] * 3
    scratch = (
        [pltpu.VMEM((idx_rows, IDXW), jnp.int32) for _ in range(3)]
        + [pltpu.VMEM((CHUNK, d), jnp.float32) for _ in range(6)]
        + [pltpu.SemaphoreType.DMA] * 4
    )

    @functools.partial(pl.kernel, out_type=out_t, mesh=mesh,
                       scratch_types=scratch,
                       compiler_params=pltpu.CompilerParams(
                           use_tc_tiling_on_sc=False))
    def k(ih0, ih1, ih2, th0, th1, th2, o0, o1, o2,
          iv0, iv1, iv2, ra0, ra1, ra2, rb0, rb1, rb2,
          g0, g1, w0, w1):
        wid = lax.axis_index("s") * NC + lax.axis_index("c")
        base = wid * per_w
        ivs = (iv0, iv1, iv2)
        ths = (th0, th1, th2)
        outs = (o0, o1, o2)
        bufs = ((ra0, ra1, ra2), (rb0, rb1, rb2))
        gsem = (g0, g1)
        wsem = (w0, w1)
        for ih, iv in zip((ih0, ih1, ih2), ivs):
            pltpu.sync_copy(ih.at[wid], iv)

        def gathers(kc, s):
            ds = []
            for th, iv, rv in zip(ths, ivs, bufs[s]):
                for u in range(SUB):
                    ds.append(pltpu.make_async_copy(
                        th.at[iv.at[kc * SUB + u]],
                        rv.at[pl.ds(u * IDXW, IDXW)], gsem[s]))
            return ds

        def writes(kc, s):
            return [pltpu.make_async_copy(
                        rv, ov.at[pl.ds(base + kc * CHUNK, CHUNK)], wsem[s])
                    for rv, ov in zip(bufs[s], outs)]

        for dsc in gathers(0, 0):
            dsc.start()

        def step(i, b):
            kc = 2 * i + b
            for dsc in gathers(kc, b):
                dsc.wait()
            drain_prev = [pltpu.make_async_copy(
                              rv, outs[j].at[pl.ds(base, CHUNK)],
                              wsem[1 - b])
                          for j, rv in enumerate(bufs[1 - b])]
            if b == 1:
                for dsc in drain_prev:
                    dsc.wait()
            else:
                @pl.when(i >= 1)
                def _():
                    for dsc in drain_prev:
                        dsc.wait()
            for dsc in gathers(kc + 1, 1 - b):
                dsc.start()
            for dsc in writes(kc, b):
                dsc.start()

        def body(i, carry):
            step(i, 0)
            step(i, 1)
            return carry

        lax.fori_loop(0, (chunks - 1) // 2, body, 0)
        for dsc in gathers(chunks - 1, 0):
            dsc.wait()
        for dsc in writes(chunks - 2, 1):
            dsc.wait()
        for dsc in writes(chunks - 1, 0):
            dsc.start()
        for dsc in writes(chunks - 1, 0):
            dsc.wait()

    return k(i0, i1, i2, t0, t1, t2)


def _tc_dense(e0, e1, e2, inter, el, w3, wT, bc, wc, bct, g1, be1, g2, be2,
              n_tokens, hd, tile):
    grid = (n_tokens // tile,)
    intd = e0.shape[1]

    def body(e0r, e1r, e2r, intr, elr, w3r, wTr, bcr, wcr, bctr, g1r, be1r,
             g2r, be2r, out):
        it = intr[...]
        e3 = jnp.where(it == 0, w3r[0:1, :],
                       jnp.where(it == 1, w3r[1:2, :], w3r[2:3, :]))
        x = jnp.concatenate([e0r[...], e1r[...], e2r[...], e3], axis=-1)
        X = lax.dot_general(x, wTr[...], (((1,), (0,)), ((), ())),
                            preferred_element_type=jnp.float32)
        X = X + bcr[...]
        mu = jnp.mean(X, axis=-1, keepdims=True)
        var = jnp.mean((X - mu) ** 2, axis=-1, keepdims=True)
        Xn = (X - mu) * lax.rsqrt(var + EPS) * g1r[...] + be1r[...]
        Y = elr[...] * wcr[...] + bctr[...]
        muY = jnp.mean(Y, axis=-1, keepdims=True)
        varY = jnp.mean((Y - muY) ** 2, axis=-1, keepdims=True)
        Yn = (Y - muY) * lax.rsqrt(varY + EPS) * g2r[...] + be2r[...]
        out[:, :hd] = Xn
        out[:, hd:] = Yn

    tok_spec = lambda w: pl.BlockSpec((tile, w), lambda i: (i, 0))
    fix_spec = lambda s: pl.BlockSpec(s, lambda i: (0, 0))
    return pl.pallas_call(
        body,
        grid=grid,
        in_specs=[tok_spec(intd)] * 3 + [tok_spec(1), tok_spec(1),
                  fix_spec((3, intd)),
                  fix_spec((4 * intd, hd))] + [fix_spec((1, hd))] * 7,
        out_specs=pl.BlockSpec((tile, 2 * hd), lambda i: (i, 0)),
        out_shape=jax.ShapeDtypeStruct((n_tokens, 2 * hd), jnp.float32),
        compiler_params=pltpu.CompilerParams(
            dimension_semantics=("arbitrary",)),
    )(e0, e1, e2, inter, el, w3, wT, bc, wc, bct, g1, be1, g2, be2)


def kernel(assessmentItemID, testId, KnowledgeTag, Interaction, elapsed,
           emb_item, emb_test, emb_tag, emb_inter,
           W_comb, b_comb, W_cont, b_cont,
           g_cat, beta_cat, g_cont, beta_cont):
    B, L = assessmentItemID.shape
    n = B * L
    intd = emb_item.shape[1]
    hd = W_comb.shape[0]
    slab = lambda a: a.reshape(NW, n // (NW * IDXW), IDXW)
    e0, e1, e2 = _sc_gather(
        slab(assessmentItemID), slab(testId), slab(KnowledgeTag),
        emb_item, emb_test, emb_tag, n, intd)
    row = lambda v: v.reshape(1, hd)
    out = _tc_dense(
        e0, e1, e2, Interaction.reshape(n, 1), elapsed.reshape(n, 1),
        emb_inter, W_comb.T, row(b_comb),
        W_cont.reshape(1, hd), row(b_cont), row(g_cat), row(beta_cat),
        row(g_cont), row(beta_cont), n, hd, tile=2048)
    return out.reshape(B, L, 2 * hd), B
